# baseline (speedup 1.0000x reference)
import functools

import jax
import jax.numpy as jnp
from jax import lax
from jax.experimental import pallas as pl
from jax.experimental.pallas import tpu as pltpu
from jax.experimental.pallas import tpu_sc as plsc

_D = 32
_DT = 4
_S = 8
_L = 128
_NL = 16
_CH = 256


def _make_lookup(num_t, bpw, num_workers, num_cores):
    ct = bpw // _L
    hpw = bpw // _CH
    nch = num_t * hpw
    mesh = plsc.VectorSubcoreMesh(core_axis_name="c", subcore_axis_name="s")

    @functools.partial(
        pl.kernel,
        out_type=jax.ShapeDtypeStruct((num_t * _DT, num_workers * ct, _S, _L),
                                      jnp.float32),
        mesh=mesh,
        scratch_types=[
            pltpu.VMEM((num_t, bpw), jnp.int32),
            pltpu.VMEM((num_t, hpw * 2, _L), jnp.int32),
            pltpu.VMEM((num_t, bpw), jnp.int32),
            pltpu.VMEM((2, _CH, _L), jnp.float32),
            pltpu.VMEM((2, _DT, hpw, _S, _L), jnp.float32),
            pltpu.SemaphoreType.DMA,
            pltpu.SemaphoreType.DMA,
        ],
        compiler_params=pltpu.CompilerParams(
            use_tc_tiling_on_sc=True, needs_layout_passes=False
        ),
    )
    def lookup(ids_hbm, table_hbm, out_hbm, idx_v, idxq_v, cb_v, rows_v,
               wbuf_v, gsem, wsem):
        wid = lax.axis_index("s") * num_cores + lax.axis_index("c")
        c0 = wid * ct
        lane = lax.iota(jnp.int32, _NL)
        pltpu.sync_copy(ids_hbm.at[:, pl.ds(wid * bpw, bpw)], idx_v)

        ngrp = bpw // _NL

        def stage_t(t, _):
            for g in range(ngrp):
                v16 = idx_v[t, pl.ds(g * _NL, _NL)]
                q = (g * _NL) // _L
                off = (g * _NL) % _L
                idxq_v[t, q, pl.ds(off, _NL)] = lax.shift_right_logical(v16, 2)
                cb_v[t, pl.ds(g * _NL, _NL)] = lax.shift_left(
                    lax.bitwise_and(v16, 3), 5
                )
            return ()

        lax.fori_loop(0, num_t, stage_t, (), unroll=False)

        def start_gather(i, buf):
            t = lax.div(i, hpw)
            h = lax.rem(i, hpw)
            handles = []
            for k in range(2):
                handles.append(pltpu.async_copy(
                    table_hbm.at[idxq_v.at[t, h * 2 + k]],
                    rows_v.at[buf, pl.ds(k * _L, _L)],
                    gsem,
                ))
            return handles

        def drain_writes(buf, t, h):
            for dt in range(_DT):
                pltpu.make_async_copy(
                    wbuf_v.at[buf, dt],
                    out_hbm.at[t * _DT + dt, pl.ds(c0 + h * hpw, hpw)],
                    wsem,
                ).wait()

        start_gather(0, 0)

        def per_chunk(i, _):
            buf = lax.rem(i, 2)
            t = lax.div(i, hpw)
            h = lax.rem(i, hpw)
            pltpu.make_async_copy(
                table_hbm.at[pl.ds(0, _CH)], rows_v.at[buf], gsem
            ).wait()

            @pl.when(i + 1 < nch)
            def _():
                start_gather(i + 1, 1 - buf)

            @pl.when(i >= 2)
            def _():
                drain_writes(buf, t, h)

            rows = rows_v.at[buf]
            for c2 in range(hpw):
                for l0 in range(0, _L, _NL):
                    j16 = lane + (c2 * _L + l0)
                    fb16 = cb_v[t, pl.ds(h * _CH + c2 * _L + l0, _NL)]
                    for d in range(_D):
                        vals = plsc.load_gather(rows, [j16, fb16 + d])
                        wbuf_v[buf, d // _S, c2, d % _S, pl.ds(l0, _NL)] = vals
            for dt in range(_DT):
                pltpu.async_copy(
                    wbuf_v.at[buf, dt],
                    out_hbm.at[t * _DT + dt, pl.ds(c0 + h * hpw, hpw)],
                    wsem,
                )
            return ()

        lax.fori_loop(0, nch, per_chunk, (), unroll=False)
        for i in (nch - 2, nch - 1):
            drain_writes(i % 2, i // hpw, i % hpw)

    return lookup


def kernel(token_ids, embeddings):
    b, t = token_ids.shape
    info = plsc.get_sparse_core_info()
    nw = info.num_cores * info.num_subcores
    bpw = b // nw
    ct = bpw // _L
    table4 = embeddings.reshape(embeddings.shape[0] // _DT, _L)
    out4 = _make_lookup(t, bpw, nw, info.num_cores)(token_ids.T, table4)
    out = (
        out4.reshape(t, _DT, nw * ct, _S, _L)
        .transpose(2, 4, 0, 1, 3)
        .reshape(b, t, _D)
    )
    return out


# speedup vs baseline: 1.0227x; 1.0227x over previous
import functools

import jax
import jax.numpy as jnp
from jax import lax
from jax.experimental import pallas as pl
from jax.experimental.pallas import tpu as pltpu
from jax.experimental.pallas import tpu_sc as plsc

_D = 32
_DT = 4
_S = 8
_L = 128
_NL = 16
_CH = 256


def _make_lookup(num_t, bpw, num_workers, num_cores):
    ct = bpw // _L
    hpw = bpw // _CH
    nch = num_t * hpw
    mesh = plsc.VectorSubcoreMesh(core_axis_name="c", subcore_axis_name="s")

    @functools.partial(
        pl.kernel,
        out_type=jax.ShapeDtypeStruct((num_t * _DT, num_workers * ct, _S, _L),
                                      jnp.float32),
        mesh=mesh,
        scratch_types=[
            pltpu.VMEM((num_t, bpw), jnp.int32),
            pltpu.VMEM((num_t, hpw * 2, _L), jnp.int32),
            pltpu.VMEM((2, _CH, _L), jnp.float32),
            pltpu.VMEM((2, _DT, hpw, _S, _L), jnp.float32),
            pltpu.SemaphoreType.DMA,
            pltpu.SemaphoreType.DMA,
        ],
        compiler_params=pltpu.CompilerParams(
            use_tc_tiling_on_sc=True, needs_layout_passes=False
        ),
    )
    def lookup(ids_hbm, table_hbm, out_hbm, idx_v, idxq_v, rows_v,
               wbuf_v, gsem, wsem):
        wid = lax.axis_index("s") * num_cores + lax.axis_index("c")
        c0 = wid * ct
        lane = lax.iota(jnp.int32, _NL)
        pltpu.sync_copy(ids_hbm.at[:, pl.ds(wid * bpw, bpw)], idx_v)

        ngrp = bpw // _NL

        def stage_t(t, _):
            for g in range(ngrp):
                v16 = idx_v[t, pl.ds(g * _NL, _NL)]
                q = (g * _NL) // _L
                off = (g * _NL) % _L
                idxq_v[t, q, pl.ds(off, _NL)] = v16
            return ()

        lax.fori_loop(0, num_t, stage_t, (), unroll=False)

        def start_gather(i, buf):
            t = lax.div(i, hpw)
            h = lax.rem(i, hpw)
            handles = []
            for k in range(2):
                handles.append(pltpu.async_copy(
                    table_hbm.at[idxq_v.at[t, h * 2 + k]],
                    rows_v.at[buf, pl.ds(k * _L, _L)],
                    gsem,
                ))
            return handles

        def drain_writes(buf, t, h):
            for dt in range(_DT):
                pltpu.make_async_copy(
                    wbuf_v.at[buf, dt],
                    out_hbm.at[t * _DT + dt, pl.ds(c0 + h * hpw, hpw)],
                    wsem,
                ).wait()

        start_gather(0, 0)

        def per_chunk(i, _):
            buf = lax.rem(i, 2)
            t = lax.div(i, hpw)
            h = lax.rem(i, hpw)
            pltpu.make_async_copy(
                table_hbm.at[pl.ds(0, _CH)], rows_v.at[buf], gsem
            ).wait()

            @pl.when(i + 1 < nch)
            def _():
                start_gather(i + 1, 1 - buf)

            @pl.when(i >= 2)
            def _():
                drain_writes(buf, t, h)

            rows = rows_v.at[buf]
            for c2 in range(hpw):
                for l0 in range(0, _L, _NL):
                    j16 = lane + (c2 * _L + l0)
                    for d in range(_D):
                        vals = plsc.load_gather(
                            rows, [j16, jnp.full((_NL,), d, jnp.int32)]
                        )
                        wbuf_v[buf, d // _S, c2, d % _S, pl.ds(l0, _NL)] = vals
            for dt in range(_DT):
                pltpu.async_copy(
                    wbuf_v.at[buf, dt],
                    out_hbm.at[t * _DT + dt, pl.ds(c0 + h * hpw, hpw)],
                    wsem,
                )
            return ()

        lax.fori_loop(0, nch, per_chunk, (), unroll=False)
        for i in (nch - 2, nch - 1):
            drain_writes(i % 2, i // hpw, i % hpw)

    return lookup


def kernel(token_ids, embeddings):
    b, t = token_ids.shape
    info = plsc.get_sparse_core_info()
    nw = info.num_cores * info.num_subcores
    bpw = b // nw
    ct = bpw // _L
    table4 = jnp.pad(embeddings, ((0, 0), (0, _L - _D)))
    out4 = _make_lookup(t, bpw, nw, info.num_cores)(token_ids.T, table4)
    out = (
        out4.reshape(t, _DT, nw * ct, _S, _L)
        .transpose(2, 4, 0, 1, 3)
        .reshape(b, t, _D)
    )
    return out


# speedup vs baseline: 1.0549x; 1.0315x over previous
import functools

import jax
import jax.numpy as jnp
from jax import lax
from jax.experimental import pallas as pl
from jax.experimental.pallas import tpu as pltpu
from jax.experimental.pallas import tpu_sc as plsc

_D = 32
_CHUNK = 1280


def _make_lookup(total, num_workers, num_cores):
    bpw = total // num_workers
    nsup = bpw // _CHUNK
    mesh = plsc.VectorSubcoreMesh(core_axis_name="c", subcore_axis_name="s")

    @functools.partial(
        pl.kernel,
        out_type=jax.ShapeDtypeStruct((total, _D), jnp.float32),
        mesh=mesh,
        scratch_types=[
            pltpu.VMEM((nsup, _CHUNK), jnp.int32),
            pltpu.VMEM((2, _CHUNK, _D), jnp.float32),
            pltpu.SemaphoreType.DMA,
            pltpu.SemaphoreType.DMA,
            pltpu.SemaphoreType.DMA,
        ],
        compiler_params=pltpu.CompilerParams(use_tc_tiling_on_sc=False),
    )
    def lookup(ids_hbm, table_hbm, out_hbm, idx_v, rows_v, gsem0, gsem1, osem):
        wid = lax.axis_index("s") * num_cores + lax.axis_index("c")
        base = wid * bpw
        gsems = (gsem0, gsem1)
        pltpu.sync_copy(ids_hbm.at[wid], idx_v)

        def start_gather(s):
            return pltpu.async_copy(
                table_hbm.at[idx_v.at[s]], rows_v.at[s % 2], gsems[s % 2]
            )

        h_g = [None] * nsup
        h_o = [None] * nsup
        h_g[0] = start_gather(0)
        for s in range(nsup):
            if s + 1 < nsup:
                if s >= 1:
                    h_o[s - 1].wait()
                h_g[s + 1] = start_gather(s + 1)
            h_g[s].wait()
            h_o[s] = pltpu.async_copy(
                rows_v.at[s % 2], out_hbm.at[pl.ds(base + s * _CHUNK, _CHUNK)], osem
            )
        h_o[nsup - 1].wait()

    return lookup


def kernel(token_ids, embeddings):
    b, t = token_ids.shape
    total = b * t
    info = plsc.get_sparse_core_info()
    nw = info.num_cores * info.num_subcores
    ids = token_ids.reshape(nw, total // (nw * _CHUNK), _CHUNK)
    out = _make_lookup(total, nw, info.num_cores)(ids, embeddings)
    return out.reshape(b, t, _D)
